# Initial kernel scaffold; baseline (speedup 1.0000x reference)
import jax
import jax.numpy as jnp
from jax.experimental import pallas as pl


def kernel(x, edge_index, lin1_W, lin1_b, blk_g, blk_b, conv_Wl, conv_bl, conv_Wr, codebooks, fin_g, fin_b, lin2_W, lin2_b):
    raise NotImplementedError("write your pallas kernel here")


# speedup vs baseline: 2.3055x; 2.3055x over previous
import functools

import jax
import jax.numpy as jnp
from jax import lax
from jax.experimental import pallas as pl
from jax.experimental.pallas import tpu as pltpu
from jax.experimental.pallas import tpu_sc as plsc

N = 10000
E = 160000
H = 256
HH = 128
K = 16
G = 3
L = 3
EPS = 1e-5

NP = 10240
EP = 163840
BN = 2048
NBLK = NP // BN

NTILE = 16
CH = 128
RT = NP // NTILE
EPT = EP // NTILE
NCH = EPT // CH
EPT2 = EP // (2 * NTILE)
NCH2 = EPT2 // CH
DW = 128


def _sc_agg_body(h0, h1, src_r, dst_r, zero_r, agg0, agg1,
                 isrc, idst, rows, acc, sem):
    c = lax.axis_index("c")
    s = lax.axis_index("s")
    pltpu.sync_copy(zero_r.at[pl.ds(s * RT, RT)], acc.at[pl.ds(s * RT, RT)])
    plsc.subcore_barrier()

    def run(h_hbm):
        def body(j, carry):
            base = s * EPT + j * CH
            pltpu.sync_copy(src_r.at[pl.ds(base, CH)], isrc)
            pltpu.sync_copy(dst_r.at[pl.ds(base, CH)], idst)
            pltpu.async_copy(h_hbm.at[isrc], rows, sem).wait()
            pltpu.sync_copy(rows, acc.at[idst], add=True)
            return carry
        lax.fori_loop(0, NCH, body, 0)

    @pl.when(c == 0)
    def _():
        run(h0)

    @pl.when(c == 1)
    def _():
        run(h1)

    plsc.subcore_barrier()

    @pl.when(c == 0)
    def _():
        pltpu.sync_copy(acc.at[pl.ds(s * RT, RT)], agg0.at[pl.ds(s * RT, RT)])

    @pl.when(c == 1)
    def _():
        pltpu.sync_copy(acc.at[pl.ds(s * RT, RT)], agg1.at[pl.ds(s * RT, RT)])


def _sc_deg_body(dst_r, ones_r, zero_r, deg_o, idst, ones_v, acc, sem):
    c = lax.axis_index("c")
    s = lax.axis_index("s")
    pltpu.sync_copy(zero_r.at[pl.ds(s * RT, RT)], acc.at[pl.ds(s * RT, RT)])
    pltpu.sync_copy(ones_r, ones_v)
    plsc.subcore_barrier()

    def body(j, carry):
        base = c * (EP // 2) + s * EPT2 + j * CH
        pltpu.sync_copy(dst_r.at[pl.ds(base, CH)], idst)
        pltpu.sync_copy(ones_v, acc.at[idst], add=True)
        return carry
    lax.fori_loop(0, NCH2, body, 0)

    plsc.subcore_barrier()
    pltpu.sync_copy(acc.at[pl.ds(s * RT, RT)], deg_o.at[c, pl.ds(s * RT, RT)])


@functools.lru_cache(maxsize=None)
def _sc_kernels():
    mesh = plsc.VectorSubcoreMesh(core_axis_name="c", subcore_axis_name="s")
    agg = pl.kernel(
        _sc_agg_body,
        out_type=[jax.ShapeDtypeStruct((NP, HH), jnp.float32),
                  jax.ShapeDtypeStruct((NP, HH), jnp.float32)],
        mesh=mesh,
        scratch_types=[
            pltpu.VMEM((CH,), jnp.int32),
            pltpu.VMEM((CH,), jnp.int32),
            pltpu.VMEM((CH, HH), jnp.float32),
            pltpu.VMEM_SHARED((NP, HH), jnp.float32),
            pltpu.SemaphoreType.DMA,
        ],
    )
    deg = pl.kernel(
        _sc_deg_body,
        out_type=jax.ShapeDtypeStruct((2, NP, DW), jnp.float32),
        mesh=mesh,
        scratch_types=[
            pltpu.VMEM((CH,), jnp.int32),
            pltpu.VMEM((CH, DW), jnp.float32),
            pltpu.VMEM_SHARED((NP, DW), jnp.float32),
            pltpu.SemaphoreType.DMA,
        ],
    )
    return agg, deg


def _fold_sum(t):
    while t.shape[-1] > 1:
        w = t.shape[-1] // 2
        t = t[..., :w] + t[..., w:]
    return t


def _ln_relu(x, g, b):
    m = _fold_sum(x) * (1.0 / H)
    xc = x - m
    v = _fold_sum(xc * xc) * (1.0 / H)
    return jnp.maximum(xc / jnp.sqrt(v + EPS) * g + b, 0.0)


def _vq(xn, cb_ref, step):
    r = xn
    c128 = lax.broadcasted_iota(jnp.int32, (BN, HH), 1)
    ids_cols = jnp.zeros((BN, HH), jnp.int32)
    iot = lax.broadcasted_iota(jnp.int32, (BN, K), 1)
    for g in range(G):
        cbg = cb_ref[g]
        csq = _fold_sum(cbg * cbg)[:, 0]
        rsum = _fold_sum(r * r)
        dsc = lax.dot_general(r, cbg, (((1,), (1,)), ((), ())),
                              preferred_element_type=jnp.float32)
        scores = (rsum - 2.0 * dsc) + csq[None, :]
        mind = jnp.min(scores, axis=1, keepdims=True)
        idx = jnp.min(jnp.where(scores <= mind, iot, K), axis=1,
                      keepdims=True)
        oh = (iot == idx).astype(jnp.float32)
        q = jnp.dot(oh, cbg, preferred_element_type=jnp.float32)
        r = r - q
        ids_cols = ids_cols + jnp.where(c128 == g, idx, 0)
    rows = lax.broadcasted_iota(jnp.int32, (BN, H), 0) + step * BN
    ssq = jnp.sum(jnp.where(rows < N, r * r, 0.0))
    return ids_cols, ssq


def _k_in(x_ref, w1t, b1, g0, b0, h0_ref, h1_ref):
    x0 = jnp.dot(x_ref[...], w1t[...],
                 preferred_element_type=jnp.float32) + b1[...]
    h = _ln_relu(x0, g0[...], b0[...])
    h0_ref[...] = h[:, :HH]
    h1_ref[...] = h[:, HH:]


def _mid_common(m0, m1, h0p, h1p, wlt, wrt, bl, cb):
    mean = jnp.concatenate([m0[...], m1[...]], axis=1)
    hprev = jnp.concatenate([h0p[...], h1p[...]], axis=1)
    xn = (jnp.dot(mean, wlt[...], preferred_element_type=jnp.float32)
          + bl[...]
          + jnp.dot(hprev, wrt[...], preferred_element_type=jnp.float32))
    i = pl.program_id(0)
    ids_cols, ssq = _vq(xn, cb, i)
    return xn, ids_cols, ssq, i


def _k_mid(m0, m1, h0p, h1p, wlt, wrt, bl, cb, g1, b1n,
           h0_ref, h1_ref, ids_ref, loss_ref):
    xn, ids_cols, ssq, i = _mid_common(m0, m1, h0p, h1p, wlt, wrt, bl, cb)
    hn = _ln_relu(xn, g1[...], b1n[...])
    h0_ref[...] = hn[:, :HH]
    h1_ref[...] = hn[:, HH:]
    ids_ref[...] = ids_cols

    @pl.when(i == 0)
    def _():
        loss_ref[...] = jnp.zeros_like(loss_ref)
    loss_ref[...] += ssq


def _k_out(m0, m1, h0p, h1p, wlt, wrt, bl, cb, fg, fb,
           y_ref, ids_ref, loss_ref):
    xn, ids_cols, ssq, i = _mid_common(m0, m1, h0p, h1p, wlt, wrt, bl, cb)
    y_ref[...] = _ln_relu(xn, fg[...], fb[...])
    ids_ref[...] = ids_cols

    @pl.when(i == 0)
    def _():
        loss_ref[...] = jnp.zeros_like(loss_ref)
    loss_ref[...] += ssq


def _k_fin(y_ref, w2t, b2, out_ref):
    out_ref[...] = jnp.dot(y_ref[...], w2t[...],
                           preferred_element_type=jnp.float32) + b2[...]


def _row_spec(cols):
    return pl.BlockSpec((BN, cols), lambda i: (i, 0))


def _full_spec(shape):
    ndim = len(shape)
    return pl.BlockSpec(shape, lambda i: (0,) * ndim)


_SPEC_H = _row_spec(HH)
_SPEC_F = _row_spec(H)
_SPEC_W = _full_spec((H, H))
_SPEC_B = _full_spec((1, H))
_SPEC_CB = _full_spec((G, K, H))
_SPEC_LOSS = pl.BlockSpec((1, 1), lambda i: (0, 0))

_OUT_HH = [jax.ShapeDtypeStruct((NP, HH), jnp.float32),
           jax.ShapeDtypeStruct((NP, HH), jnp.float32)]
_OUT_IDSLOSS = [jax.ShapeDtypeStruct((NP, HH), jnp.int32),
                jax.ShapeDtypeStruct((1, 1), jnp.float32)]

_in_call = pl.pallas_call(
    _k_in,
    grid=(NBLK,),
    in_specs=[_SPEC_F, _SPEC_W, _SPEC_B, _SPEC_B, _SPEC_B],
    out_specs=[_SPEC_H, _SPEC_H],
    out_shape=_OUT_HH,
)

_mid_call = pl.pallas_call(
    _k_mid,
    grid=(NBLK,),
    in_specs=[_SPEC_H, _SPEC_H, _SPEC_H, _SPEC_H, _SPEC_W, _SPEC_W,
              _SPEC_B, _SPEC_CB, _SPEC_B, _SPEC_B],
    out_specs=[_SPEC_H, _SPEC_H, _row_spec(HH), _SPEC_LOSS],
    out_shape=_OUT_HH + _OUT_IDSLOSS,
)

_out_call = pl.pallas_call(
    _k_out,
    grid=(NBLK,),
    in_specs=[_SPEC_H, _SPEC_H, _SPEC_H, _SPEC_H, _SPEC_W, _SPEC_W,
              _SPEC_B, _SPEC_CB, _SPEC_B, _SPEC_B],
    out_specs=[_SPEC_F, _row_spec(HH), _SPEC_LOSS],
    out_shape=[jax.ShapeDtypeStruct((NP, H), jnp.float32)] + _OUT_IDSLOSS,
)

_fin_call = pl.pallas_call(
    _k_fin,
    grid=(NBLK,),
    in_specs=[_SPEC_F, _SPEC_W, _SPEC_B],
    out_specs=_SPEC_F,
    out_shape=jax.ShapeDtypeStruct((NP, H), jnp.float32),
)


def kernel(x, edge_index, lin1_W, lin1_b, blk_g, blk_b, conv_Wl, conv_bl,
           conv_Wr, codebooks, fin_g, fin_b, lin2_W, lin2_b):
    f32 = jnp.float32
    xp = jnp.pad(x, ((0, NP - N), (0, 0)))
    src = jnp.pad(edge_index[0], (0, EP - E))
    dst = jnp.pad(edge_index[1], (0, EP - E), constant_values=N)

    zeros128 = jnp.zeros((NP, HH), f32)
    zeros_dw = jnp.zeros((NP, DW), f32)
    ones_dw = jnp.ones((CH, DW), f32)

    _sc_agg, _sc_deg = _sc_kernels()
    degp = _sc_deg(dst, ones_dw, zeros_dw)
    degc = jnp.clip(degp[0, :, 0] + degp[1, :, 0], 1.0)[:, None]

    h0, h1 = _in_call(xp, lin1_W.T, lin1_b[None, :], blk_g[0:1], blk_b[0:1])

    losses = []
    ids_list = []
    for i in range(L):
        agg0, agg1 = _sc_agg(h0, h1, src, dst, zeros128)
        m0 = agg0 / degc
        m1 = agg1 / degc
        wlt = conv_Wl[i].T
        wrt = conv_Wr[i].T
        bl = conv_bl[i][None, :]
        cb = codebooks[i]
        if i < L - 1:
            h0n, h1n, ids_i, l_i = _mid_call(
                m0, m1, h0, h1, wlt, wrt, bl, cb,
                blk_g[i + 1:i + 2], blk_b[i + 1:i + 2])
            h0, h1 = h0n, h1n
        else:
            y, ids_i, l_i = _out_call(
                m0, m1, h0, h1, wlt, wrt, bl, cb, fin_g[None, :],
                fin_b[None, :])
        ids_list.append(ids_i[:N, :G])
        losses.append(l_i[0, 0])

    out_p = _fin_call(y, lin2_W.T, lin2_b[None, :])
    total_loss = (losses[0] + losses[1] + losses[2]) / (N * H)
    ids = jnp.concatenate(ids_list, axis=1)
    return out_p[:N], total_loss, ids
